# baseline (speedup 1.0000x reference)
import functools

import jax
import jax.numpy as jnp
from jax import lax
from jax.experimental import pallas as pl
from jax.experimental.pallas import tpu as pltpu
from jax.experimental.pallas import tpu_sc as plsc

VOCAB = 100000
D = 64
WPAD = 128
BATCH = 4096
SEQ = 50
LANES = 16

NC = 2
NS = 16
NW = NC * NS

BATCH_W = BATCH // NW
SEQ_CHUNK = 4
CHUNK = SEQ_CHUNK * SEQ
DMA_ROWS = 40
Q = CHUNK // DMA_ROWS
NCHUNK = BATCH_W // SEQ_CHUNK
NTOT_CHUNK = BATCH * SEQ // CHUNK
NBUF = 2


def _pos_table():
    pos = jnp.arange(SEQ, dtype=jnp.float32)[:, None]
    i = jnp.arange(D, dtype=jnp.float32)[None, :]
    angle = pos / jnp.power(10000.0, 2.0 * jnp.floor(i / 2.0) / D)
    angle = angle.at[:, 0::2].set(jnp.sin(angle[:, 0::2]))
    angle = angle.at[:, 1::2].set(jnp.cos(angle[:, 1::2]))
    return angle


def _sc_embed(wpack, idx3d, comb):
    mesh = plsc.VectorSubcoreMesh(core_axis_name="c", subcore_axis_name="s")

    @functools.partial(
        pl.kernel,
        mesh=mesh,
        compiler_params=pltpu.CompilerParams(use_tc_tiling_on_sc=True),
        out_type=jax.ShapeDtypeStruct((BATCH, SEQ, D), jnp.float32),
        scratch_types=[
            pltpu.VMEM((NBUF, Q, DMA_ROWS), jnp.int32),
            pltpu.VMEM((NBUF, SEQ, WPAD), jnp.float32),
            pltpu.VMEM((NBUF, CHUNK, WPAD), jnp.float32),
            pltpu.VMEM((NBUF, SEQ_CHUNK, SEQ, D), jnp.float32),
            pltpu.SemaphoreType.DMA((NBUF,)),
            pltpu.SemaphoreType.DMA((NBUF,)),
            pltpu.SemaphoreType.DMA((NBUF,)),
        ],
    )
    def k(w_hbm, idx_hbm, comb_hbm, out_hbm, idx_v, comb_v, gbuf,
          wbuf, isem, gsem, wsem):
        wid = lax.axis_index("s") * NC + lax.axis_index("c")
        out_w = wid * BATCH_W
        idx_w = wid * NCHUNK

        def stage_idx(c, b):
            pltpu.async_copy(idx_hbm.at[idx_w + c], idx_v.at[b], isem.at[b])
            pltpu.async_copy(comb_hbm.at[idx_w + c], comb_v.at[b],
                             isem.at[b])

        def fire_gathers(b):
            pltpu.make_async_copy(idx_hbm.at[pl.ds(0, 1)],
                                  idx_v.at[b], isem.at[b]).wait()
            pltpu.make_async_copy(comb_hbm.at[pl.ds(0, 1)],
                                  comb_v.at[b], isem.at[b]).wait()
            for q in range(Q):
                pltpu.async_copy(
                    w_hbm.at[idx_v.at[b].at[q]],
                    gbuf.at[b].at[pl.ds(q * DMA_ROWS, DMA_ROWS)],
                    gsem.at[b])

        for b in range(NBUF):
            stage_idx(b, b)
        for b in range(NBUF):
            fire_gathers(b)

        def pair_body(i, carry):
            for b in range(NBUF):
                c = i * NBUF + b
                pltpu.make_async_copy(w_hbm.at[pl.ds(0, CHUNK)], gbuf.at[b],
                                      gsem.at[b]).wait()

                @pl.when(i > 0)
                def _():
                    pltpu.make_async_copy(wbuf.at[b],
                                          out_hbm.at[pl.ds(0, SEQ_CHUNK)],
                                          wsem.at[b]).wait()

                @plsc.parallel_loop(0, SEQ, unroll=5)
                def _(s):
                    pv = [comb_v[b, s, pl.ds(D + j * LANES, LANES)]
                          for j in range(D // LANES)]
                    for t in range(SEQ_CHUNK):
                        r = t * SEQ + s
                        a = comb_v[b, s, pl.ds(t * LANES, LANES)]
                        bc = 0.125 - a
                        for j in range(D // LANES):
                            sl = pl.ds(j * LANES, LANES)
                            hs = pl.ds(D + j * LANES, LANES)
                            wbuf[b, t, s, sl] = (
                                gbuf[b, r, sl] * bc + gbuf[b, r, hs] * a
                                + pv[j])

                @pl.when(i < NCHUNK // NBUF - 1)
                def _():
                    stage_idx(c + NBUF, b)

                pltpu.async_copy(
                    wbuf.at[b],
                    out_hbm.at[pl.ds(out_w + c * SEQ_CHUNK, SEQ_CHUNK)],
                    wsem.at[b])

                @pl.when(i < NCHUNK // NBUF - 1)
                def _():
                    fire_gathers(b)
            return carry

        lax.fori_loop(0, NCHUNK // NBUF, pair_body, 0)

        for b in range(NBUF):
            pltpu.make_async_copy(wbuf.at[b], out_hbm.at[pl.ds(0, SEQ_CHUNK)],
                                  wsem.at[b]).wait()

    return k(wpack, idx3d, comb)


def kernel(input, weight):
    wpack = weight.reshape(VOCAB // 2, WPAD)
    flat = input.reshape(BATCH * SEQ)
    idx3d = (flat >> 1).reshape(NTOT_CHUNK, Q, DMA_ROWS)
    bits = ((flat & 1).astype(jnp.float32) * 0.125).reshape(
        NTOT_CHUNK, SEQ_CHUNK, SEQ)
    acoef = jnp.broadcast_to(
        jnp.transpose(bits, (0, 2, 1))[..., None],
        (NTOT_CHUNK, SEQ, SEQ_CHUNK, LANES)).reshape(NTOT_CHUNK, SEQ, D)
    posb = jnp.broadcast_to(_pos_table()[None], (NTOT_CHUNK, SEQ, D))
    comb = jnp.concatenate([acoef, posb], axis=-1)
    return _sc_embed(wpack, idx3d, comb)
